# baseline (speedup 1.0000x reference)
import functools

import jax
import jax.numpy as jnp
from jax import lax
from jax.experimental import pallas as pl
from jax.experimental.pallas import tpu as pltpu
from jax.experimental.pallas import tpu_sc as plsc

N = 100000
E = 3200000
D = 16
KSTEPS = 3
ALPHA = 0.5

NC, NS = 2, 16
LANES = 128

NP = 100352
R8 = NP // 8
EP = 3211264
EROWS = EP // LANES
NODE_ROWS_PER_TILE = NP // NS

SP_ROWS_PER_SC = EROWS // NC
SP_ROWS_PER_TILE = SP_ROWS_PER_SC // NS
GRP = 8
SP_GROUPS = SP_ROWS_PER_TILE // GRP
DGRP = 8

SP_SUPER_PER_SC = SP_ROWS_PER_SC // GRP

DG_ROWS_PER_TILE = EROWS // NS
DG_GROUPS = DG_ROWS_PER_TILE // DGRP

_MESH = plsc.VectorSubcoreMesh(core_axis_name="c", subcore_axis_name="s")
_SC_PARAMS = pltpu.CompilerParams(use_tc_tiling_on_sc=False)


def _zero_fill(buf_ref, nrows):
    @pl.loop(0, nrows)
    def _(i):
        buf_ref[i, :] = jnp.zeros((D,), jnp.float32)


def _zero_acc(acc_ref, zbuf_ref, node_base):
    @pl.loop(0, 6)
    def _(q):
        pltpu.sync_copy(zbuf_ref, acc_ref.at[pl.ds(node_base + q * 1024, 1024)])
    pltpu.sync_copy(zbuf_ref.at[pl.ds(0, 128)],
                    acc_ref.at[pl.ds(node_base + 6144, 128)])


def _deg_body(ei_ref, deg_ref, idx_ref, ones_ref, acc_ref, isem, ssem):
    c = lax.axis_index("c")
    s = lax.axis_index("s")
    node_base = s * NODE_ROWS_PER_TILE

    _zero_fill(ones_ref, DGRP * LANES)
    _zero_acc(acc_ref, ones_ref, node_base)

    @pl.loop(0, DGRP * LANES)
    def _(i):
        ones_ref[i, :] = jnp.full((D,), 1.0, jnp.float32)

    plsc.subcore_barrier()

    row_base = s * DG_ROWS_PER_TILE

    def scatter_wait():
        for j in range(DGRP):
            pltpu.make_async_copy(
                ones_ref.at[pl.ds(j * LANES, LANES)],
                acc_ref.at[idx_ref.at[0, 0]], ssem).wait()

    pltpu.async_copy(ei_ref.at[c, pl.ds(row_base, DGRP)], idx_ref.at[0], isem)

    @pl.loop(0, DG_GROUPS)
    def _(g):
        p = lax.rem(g, 2)
        pltpu.make_async_copy(
            ei_ref.at[c, pl.ds(row_base + g * DGRP, DGRP)],
            idx_ref.at[p], isem).wait()
        for j in range(DGRP):
            pltpu.async_copy(ones_ref.at[pl.ds(j * LANES, LANES)],
                             acc_ref.at[idx_ref.at[p, j]], ssem, add=True)

        @pl.when(g >= 1)
        def _():
            scatter_wait()

        @pl.when(g + 1 < DG_GROUPS)
        def _():
            pltpu.async_copy(ei_ref.at[c, pl.ds(row_base + (g + 1) * DGRP, DGRP)],
                             idx_ref.at[1 - p], isem)

    scatter_wait()
    plsc.subcore_barrier()
    pltpu.sync_copy(acc_ref.at[pl.ds(node_base, NODE_ROWS_PER_TILE)],
                    deg_ref.at[c, pl.ds(node_base, NODE_ROWS_PER_TILE)])


_deg_kernel = functools.partial(
    pl.kernel,
    out_type=jax.ShapeDtypeStruct((NC, NP, D), jnp.float32),
    mesh=_MESH,
    scratch_types=[
        pltpu.VMEM((2, DGRP, LANES), jnp.int32),
        pltpu.VMEM((DGRP * LANES, D), jnp.float32),
        pltpu.VMEM_SHARED((NP, D), jnp.float32),
        pltpu.SemaphoreType.DMA,
        pltpu.SemaphoreType.DMA,
    ],
    compiler_params=_SC_PARAMS,
)(_deg_body)


def _spmm_body(ei_ref, u_ref, out_ref, idxs_ref, idxd_ref, rows_ref, acc_ref,
               isem, gsem, ssem):
    c = lax.axis_index("c")
    s = lax.axis_index("s")
    node_base = s * NODE_ROWS_PER_TILE

    _zero_fill(rows_ref, GRP * LANES)
    _zero_acc(acc_ref, rows_ref, node_base)
    plsc.subcore_barrier()

    row_base = c * SP_SUPER_PER_SC + s * SP_GROUPS

    pltpu.async_copy(ei_ref.at[0, pl.ds(row_base, 1)], idxs_ref.at[0], isem)
    pltpu.async_copy(ei_ref.at[1, pl.ds(row_base, 1)], idxd_ref.at[0], isem)

    @pl.loop(0, SP_GROUPS)
    def _(g):
        p = lax.rem(g, 2)
        pltpu.make_async_copy(
            ei_ref.at[0, pl.ds(row_base + g, 1)],
            idxs_ref.at[p], isem).wait()
        pltpu.make_async_copy(
            ei_ref.at[1, pl.ds(row_base + g, 1)],
            idxd_ref.at[p], isem).wait()

        gat = pltpu.async_copy(u_ref.at[idxs_ref.at[p, 0]], rows_ref, gsem)

        @pl.when(g + 1 < SP_GROUPS)
        def _():
            pltpu.async_copy(
                ei_ref.at[0, pl.ds(row_base + (g + 1), 1)],
                idxs_ref.at[1 - p], isem)

        gat.wait()
        sct = pltpu.async_copy(rows_ref, acc_ref.at[idxd_ref.at[p, 0]], ssem,
                               add=True)

        @pl.when(g + 1 < SP_GROUPS)
        def _():
            pltpu.async_copy(
                ei_ref.at[1, pl.ds(row_base + (g + 1), 1)],
                idxd_ref.at[1 - p], isem)

        sct.wait()

    plsc.subcore_barrier()
    pltpu.sync_copy(acc_ref.at[pl.ds(node_base, NODE_ROWS_PER_TILE)],
                    out_ref.at[c, pl.ds(node_base, NODE_ROWS_PER_TILE)])


_spmm_kernel = functools.partial(
    pl.kernel,
    out_type=jax.ShapeDtypeStruct((NC, NP, D), jnp.float32),
    mesh=_MESH,
    scratch_types=[
        pltpu.VMEM((2, 1, GRP * LANES), jnp.int32),
        pltpu.VMEM((2, 1, GRP * LANES), jnp.int32),
        pltpu.VMEM((GRP * LANES, D), jnp.float32),
        pltpu.VMEM_SHARED((NP, D), jnp.float32),
        pltpu.SemaphoreType.DMA,
        pltpu.SemaphoreType.DMA,
        pltpu.SemaphoreType.DMA,
    ],
    compiler_params=_SC_PARAMS,
)(_spmm_body)


_MM_BLK = 2000


def _mm_body(x_ref, w_ref, b_ref, o_ref):
    o_ref[...] = lax.dot_general(
        x_ref[...], w_ref[...], (((1,), (1,)), ((), ())),
        precision=lax.Precision.HIGHEST) + b_ref[...]


def _matmul(x, w, b2):
    return pl.pallas_call(
        _mm_body,
        grid=(N // _MM_BLK,),
        in_specs=[
            pl.BlockSpec((_MM_BLK, 128), lambda i: (i, 0)),
            pl.BlockSpec((D, 128), lambda i: (0, 0)),
            pl.BlockSpec((1, D), lambda i: (0, 0)),
        ],
        out_specs=pl.BlockSpec((_MM_BLK, D), lambda i: (i, 0)),
        out_shape=jax.ShapeDtypeStruct((N, D), jnp.float32),
    )(x, w, b2)


_EW_BLK = 1792


def _prep_body(degs_ref, degd_ref, h0_ref, on_ref, in_ref, u0_ref):
    on = lax.rsqrt(jnp.maximum(degs_ref[0], 1.0))
    inn = lax.rsqrt(jnp.maximum(degd_ref[0], 1.0))
    on_ref[...] = on
    in_ref[...] = inn
    u0_ref[...] = h0_ref[...] * on


def _prep(deg8, h08):
    return pl.pallas_call(
        _prep_body,
        grid=(R8 // _EW_BLK,),
        in_specs=[
            pl.BlockSpec((1, _EW_BLK, 128), lambda i: (0, i, 0)),
            pl.BlockSpec((1, _EW_BLK, 128), lambda i: (1, i, 0)),
            pl.BlockSpec((_EW_BLK, 128), lambda i: (i, 0)),
        ],
        out_specs=[pl.BlockSpec((_EW_BLK, 128), lambda i: (i, 0))] * 3,
        out_shape=[jax.ShapeDtypeStruct((R8, 128), jnp.float32)] * 3,
    )(deg8, deg8, h08)


def _combine_mid_body(p0_ref, p1_ref, in_ref, on_ref, h0_ref, u_ref):
    agg = (p0_ref[0] + p1_ref[0]) * in_ref[...]
    u_ref[...] = ((1.0 - ALPHA) * agg + ALPHA * h0_ref[...]) * on_ref[...]


def _combine_final_body(p0_ref, p1_ref, in_ref, h0_ref, h_ref):
    agg = (p0_ref[0] + p1_ref[0]) * in_ref[...]
    h_ref[...] = (1.0 - ALPHA) * agg + ALPHA * h0_ref[...]


def _combine_mid(parts8, inn, onn, h08):
    return pl.pallas_call(
        _combine_mid_body,
        grid=(R8 // _EW_BLK,),
        in_specs=[
            pl.BlockSpec((1, _EW_BLK, 128), lambda i: (0, i, 0)),
            pl.BlockSpec((1, _EW_BLK, 128), lambda i: (1, i, 0)),
            pl.BlockSpec((_EW_BLK, 128), lambda i: (i, 0)),
            pl.BlockSpec((_EW_BLK, 128), lambda i: (i, 0)),
            pl.BlockSpec((_EW_BLK, 128), lambda i: (i, 0)),
        ],
        out_specs=pl.BlockSpec((_EW_BLK, 128), lambda i: (i, 0)),
        out_shape=jax.ShapeDtypeStruct((R8, 128), jnp.float32),
    )(parts8, parts8, inn, onn, h08)


def _combine_final(parts8, inn, h08):
    return pl.pallas_call(
        _combine_final_body,
        grid=(R8 // _EW_BLK,),
        in_specs=[
            pl.BlockSpec((1, _EW_BLK, 128), lambda i: (0, i, 0)),
            pl.BlockSpec((1, _EW_BLK, 128), lambda i: (1, i, 0)),
            pl.BlockSpec((_EW_BLK, 128), lambda i: (i, 0)),
            pl.BlockSpec((_EW_BLK, 128), lambda i: (i, 0)),
        ],
        out_specs=pl.BlockSpec((_EW_BLK, 128), lambda i: (i, 0)),
        out_shape=jax.ShapeDtypeStruct((R8, 128), jnp.float32),
    )(parts8, parts8, inn, h08)


def kernel(in_feat, edge_index, W, b):
    ei32 = edge_index.astype(jnp.int32)
    pad = jnp.full((2, EP - E), N, jnp.int32)
    ei = jnp.concatenate([ei32, pad], axis=1).reshape(2, EROWS, LANES)

    h0 = _matmul(in_feat, W, b.reshape(1, D))
    h08 = jnp.pad(h0, ((0, NP - N), (0, 0))).reshape(R8, 128)

    deg = _deg_kernel(ei)
    onn, inn, u8 = _prep(deg.reshape(NC, R8, 128), h08)

    ei_w = ei.reshape(2, EROWS // GRP, GRP * LANES)
    for k in range(KSTEPS):
        parts = _spmm_kernel(ei_w, u8.reshape(NP, D))
        parts8 = parts.reshape(NC, R8, 128)
        if k < KSTEPS - 1:
            u8 = _combine_mid(parts8, inn, onn, h08)
        else:
            h8 = _combine_final(parts8, inn, h08)
    return h8.reshape(NP, D)[:N]


# speedup vs baseline: 1.2611x; 1.2611x over previous
import functools

import jax
import jax.numpy as jnp
from jax import lax
from jax.experimental import pallas as pl
from jax.experimental.pallas import tpu as pltpu
from jax.experimental.pallas import tpu_sc as plsc

N = 100000
E = 3200000
D = 16
KSTEPS = 3
ALPHA = 0.5

NC, NS = 2, 16
LANES = 128

NP = 100352
R8 = NP // 8
EROWS = E // LANES
NODE_ROWS_PER_TILE = NP // NS

GRP = 8

SP_ROWS_PER_SC = EROWS // NC
SP_FULL_GROUPS = 97
SP_TAIL_BIG, SP_TAIL_SMALL = 6, 5
SP_EXTRA_TILES = 4

DG_FULL_GROUPS = 195
DG_TAIL_BIG, DG_TAIL_SMALL = 3, 2
DG_EXTRA_TILES = 8

_MESH = plsc.VectorSubcoreMesh(core_axis_name="c", subcore_axis_name="s")
_SC_PARAMS = pltpu.CompilerParams(use_tc_tiling_on_sc=False)


def _zero_fill(buf_ref, nrows):
    @pl.loop(0, nrows)
    def _(i):
        buf_ref[i, :] = jnp.zeros((D,), jnp.float32)


def _zero_acc(acc_ref, zbuf_ref, node_base):
    @pl.loop(0, 6)
    def _(q):
        pltpu.sync_copy(zbuf_ref, acc_ref.at[pl.ds(node_base + q * 1024, 1024)])
    pltpu.sync_copy(zbuf_ref.at[pl.ds(0, 128)],
                    acc_ref.at[pl.ds(node_base + 6144, 128)])


def _deg_body(ei_ref, deg_ref, idx_ref, ones_ref, acc_ref, isem, ssem):
    c = lax.axis_index("c")
    s = lax.axis_index("s")
    node_base = s * NODE_ROWS_PER_TILE

    _zero_fill(ones_ref, GRP * LANES)
    _zero_acc(acc_ref, ones_ref, node_base)

    @pl.loop(0, GRP * LANES)
    def _(i):
        ones_ref[i, :] = jnp.full((D,), 1.0, jnp.float32)

    plsc.subcore_barrier()

    row_base = s * 1562 + jnp.minimum(s, DG_EXTRA_TILES)

    def scatter_wait():
        for j in range(GRP):
            pltpu.make_async_copy(
                ones_ref.at[pl.ds(j * LANES, LANES)],
                acc_ref.at[idx_ref.at[0, 0]], ssem).wait()

    pltpu.async_copy(ei_ref.at[c, pl.ds(row_base, GRP)], idx_ref.at[0], isem)

    @pl.loop(0, DG_FULL_GROUPS)
    def _(g):
        p = lax.rem(g, 2)
        pltpu.make_async_copy(
            ei_ref.at[c, pl.ds(row_base + g * GRP, GRP)],
            idx_ref.at[p], isem).wait()
        for j in range(GRP):
            pltpu.async_copy(ones_ref.at[pl.ds(j * LANES, LANES)],
                             acc_ref.at[idx_ref.at[p, j]], ssem, add=True)

        @pl.when(g >= 1)
        def _():
            scatter_wait()

        @pl.when(g + 1 < DG_FULL_GROUPS)
        def _():
            pltpu.async_copy(ei_ref.at[c, pl.ds(row_base + (g + 1) * GRP, GRP)],
                             idx_ref.at[1 - p], isem)

    scatter_wait()

    tail_base = row_base + DG_FULL_GROUPS * GRP

    def deg_tail(ntail):
        pltpu.sync_copy(ei_ref.at[c, pl.ds(tail_base, ntail)],
                        idx_ref.at[0, pl.ds(0, ntail)])
        tails = []
        for j in range(ntail):
            tails.append(pltpu.async_copy(
                ones_ref.at[pl.ds(j * LANES, LANES)],
                acc_ref.at[idx_ref.at[0, j]], ssem, add=True))
        for cp in tails:
            cp.wait()

    @pl.when(s < DG_EXTRA_TILES)
    def _():
        deg_tail(DG_TAIL_BIG)

    @pl.when(s >= DG_EXTRA_TILES)
    def _():
        deg_tail(DG_TAIL_SMALL)

    plsc.subcore_barrier()
    pltpu.sync_copy(acc_ref.at[pl.ds(node_base, NODE_ROWS_PER_TILE)],
                    deg_ref.at[c, pl.ds(node_base, NODE_ROWS_PER_TILE)])


_deg_kernel = functools.partial(
    pl.kernel,
    out_type=jax.ShapeDtypeStruct((NC, NP, D), jnp.float32),
    mesh=_MESH,
    scratch_types=[
        pltpu.VMEM((2, GRP, LANES), jnp.int32),
        pltpu.VMEM((GRP * LANES, D), jnp.float32),
        pltpu.VMEM_SHARED((NP, D), jnp.float32),
        pltpu.SemaphoreType.DMA,
        pltpu.SemaphoreType.DMA,
    ],
    compiler_params=_SC_PARAMS,
)(_deg_body)


def _spmm_body(ei_ref, u_ref, out_ref, idxs_ref, idxd_ref, rows_ref, acc_ref,
               isem, gsem, ssem):
    c = lax.axis_index("c")
    s = lax.axis_index("s")
    node_base = s * NODE_ROWS_PER_TILE

    _zero_fill(rows_ref, GRP * LANES)
    _zero_acc(acc_ref, rows_ref, node_base)
    plsc.subcore_barrier()

    row_base = c * SP_ROWS_PER_SC + s * 781 + jnp.minimum(s, SP_EXTRA_TILES)

    pltpu.async_copy(ei_ref.at[0, pl.ds(row_base, GRP)], idxs_ref.at[0], isem)
    pltpu.async_copy(ei_ref.at[1, pl.ds(row_base, GRP)], idxd_ref.at[0], isem)

    @pl.loop(0, SP_FULL_GROUPS)
    def _(g):
        p = lax.rem(g, 2)
        pltpu.make_async_copy(
            ei_ref.at[0, pl.ds(row_base + g * GRP, GRP)],
            idxs_ref.at[p], isem).wait()
        pltpu.make_async_copy(
            ei_ref.at[1, pl.ds(row_base + g * GRP, GRP)],
            idxd_ref.at[p], isem).wait()

        gathers = []
        for j in range(GRP):
            gathers.append(pltpu.async_copy(
                u_ref.at[idxs_ref.at[p, j]],
                rows_ref.at[pl.ds(j * LANES, LANES)], gsem))

        @pl.when(g + 1 < SP_FULL_GROUPS)
        def _():
            pltpu.async_copy(
                ei_ref.at[0, pl.ds(row_base + (g + 1) * GRP, GRP)],
                idxs_ref.at[1 - p], isem)

        scatters = []
        for j in range(GRP):
            gathers[j].wait()
            scatters.append(pltpu.async_copy(
                rows_ref.at[pl.ds(j * LANES, LANES)],
                acc_ref.at[idxd_ref.at[p, j]], ssem, add=True))

        @pl.when(g + 1 < SP_FULL_GROUPS)
        def _():
            pltpu.async_copy(
                ei_ref.at[1, pl.ds(row_base + (g + 1) * GRP, GRP)],
                idxd_ref.at[1 - p], isem)

        for cp in scatters:
            cp.wait()

    tail_base = row_base + SP_FULL_GROUPS * GRP

    def sp_tail(ntail):
        pltpu.sync_copy(ei_ref.at[0, pl.ds(tail_base, ntail)],
                        idxs_ref.at[0, pl.ds(0, ntail)])
        pltpu.sync_copy(ei_ref.at[1, pl.ds(tail_base, ntail)],
                        idxd_ref.at[0, pl.ds(0, ntail)])
        tg = []
        for j in range(ntail):
            tg.append(pltpu.async_copy(
                u_ref.at[idxs_ref.at[0, j]],
                rows_ref.at[pl.ds(j * LANES, LANES)], gsem))
        ts = []
        for j in range(ntail):
            tg[j].wait()
            ts.append(pltpu.async_copy(
                rows_ref.at[pl.ds(j * LANES, LANES)],
                acc_ref.at[idxd_ref.at[0, j]], ssem, add=True))
        for cp in ts:
            cp.wait()

    @pl.when(s < SP_EXTRA_TILES)
    def _():
        sp_tail(SP_TAIL_BIG)

    @pl.when(s >= SP_EXTRA_TILES)
    def _():
        sp_tail(SP_TAIL_SMALL)

    plsc.subcore_barrier()
    pltpu.sync_copy(acc_ref.at[pl.ds(node_base, NODE_ROWS_PER_TILE)],
                    out_ref.at[c, pl.ds(node_base, NODE_ROWS_PER_TILE)])


_spmm_kernel = functools.partial(
    pl.kernel,
    out_type=jax.ShapeDtypeStruct((NC, NP, D), jnp.float32),
    mesh=_MESH,
    scratch_types=[
        pltpu.VMEM((2, GRP, LANES), jnp.int32),
        pltpu.VMEM((2, GRP, LANES), jnp.int32),
        pltpu.VMEM((GRP * LANES, D), jnp.float32),
        pltpu.VMEM_SHARED((NP, D), jnp.float32),
        pltpu.SemaphoreType.DMA,
        pltpu.SemaphoreType.DMA,
        pltpu.SemaphoreType.DMA,
    ],
    compiler_params=_SC_PARAMS,
)(_spmm_body)


_MM_BLK = 2000


def _mm_body(x_ref, w_ref, b_ref, o_ref):
    o_ref[...] = lax.dot_general(
        x_ref[...], w_ref[...], (((1,), (1,)), ((), ())),
        precision=lax.Precision.HIGHEST) + b_ref[...]


def _matmul(x, w, b2):
    return pl.pallas_call(
        _mm_body,
        grid=(N // _MM_BLK,),
        in_specs=[
            pl.BlockSpec((_MM_BLK, 128), lambda i: (i, 0)),
            pl.BlockSpec((D, 128), lambda i: (0, 0)),
            pl.BlockSpec((1, D), lambda i: (0, 0)),
        ],
        out_specs=pl.BlockSpec((_MM_BLK, D), lambda i: (i, 0)),
        out_shape=jax.ShapeDtypeStruct((N, D), jnp.float32),
    )(x, w, b2)


_EW_BLK = 1792


def _prep_body(degs_ref, degd_ref, h0_ref, on_ref, in_ref, u0_ref):
    on = lax.rsqrt(jnp.maximum(degs_ref[0], 1.0))
    inn = lax.rsqrt(jnp.maximum(degd_ref[0], 1.0))
    on_ref[...] = on
    in_ref[...] = inn
    u0_ref[...] = h0_ref[...] * on


def _prep(deg8, h08):
    return pl.pallas_call(
        _prep_body,
        grid=(R8 // _EW_BLK,),
        in_specs=[
            pl.BlockSpec((1, _EW_BLK, 128), lambda i: (0, i, 0)),
            pl.BlockSpec((1, _EW_BLK, 128), lambda i: (1, i, 0)),
            pl.BlockSpec((_EW_BLK, 128), lambda i: (i, 0)),
        ],
        out_specs=[pl.BlockSpec((_EW_BLK, 128), lambda i: (i, 0))] * 3,
        out_shape=[jax.ShapeDtypeStruct((R8, 128), jnp.float32)] * 3,
    )(deg8, deg8, h08)


def _combine_mid_body(p0_ref, p1_ref, in_ref, on_ref, h0_ref, u_ref):
    agg = (p0_ref[0] + p1_ref[0]) * in_ref[...]
    u_ref[...] = ((1.0 - ALPHA) * agg + ALPHA * h0_ref[...]) * on_ref[...]


def _combine_final_body(p0_ref, p1_ref, in_ref, h0_ref, h_ref):
    agg = (p0_ref[0] + p1_ref[0]) * in_ref[...]
    h_ref[...] = (1.0 - ALPHA) * agg + ALPHA * h0_ref[...]


def _combine_mid(parts8, inn, onn, h08):
    return pl.pallas_call(
        _combine_mid_body,
        grid=(R8 // _EW_BLK,),
        in_specs=[
            pl.BlockSpec((1, _EW_BLK, 128), lambda i: (0, i, 0)),
            pl.BlockSpec((1, _EW_BLK, 128), lambda i: (1, i, 0)),
            pl.BlockSpec((_EW_BLK, 128), lambda i: (i, 0)),
            pl.BlockSpec((_EW_BLK, 128), lambda i: (i, 0)),
            pl.BlockSpec((_EW_BLK, 128), lambda i: (i, 0)),
        ],
        out_specs=pl.BlockSpec((_EW_BLK, 128), lambda i: (i, 0)),
        out_shape=jax.ShapeDtypeStruct((R8, 128), jnp.float32),
    )(parts8, parts8, inn, onn, h08)


def _combine_final(parts8, inn, h08):
    return pl.pallas_call(
        _combine_final_body,
        grid=(R8 // _EW_BLK,),
        in_specs=[
            pl.BlockSpec((1, _EW_BLK, 128), lambda i: (0, i, 0)),
            pl.BlockSpec((1, _EW_BLK, 128), lambda i: (1, i, 0)),
            pl.BlockSpec((_EW_BLK, 128), lambda i: (i, 0)),
            pl.BlockSpec((_EW_BLK, 128), lambda i: (i, 0)),
        ],
        out_specs=pl.BlockSpec((_EW_BLK, 128), lambda i: (i, 0)),
        out_shape=jax.ShapeDtypeStruct((R8, 128), jnp.float32),
    )(parts8, parts8, inn, h08)


def kernel(in_feat, edge_index, W, b):
    ei = edge_index.astype(jnp.int32).reshape(2, EROWS, LANES)

    h0 = _matmul(in_feat, W, b.reshape(1, D))
    h08 = jnp.pad(h0, ((0, NP - N), (0, 0))).reshape(R8, 128)

    deg = _deg_kernel(ei)
    onn, inn, u8 = _prep(deg.reshape(NC, R8, 128), h08)

    for k in range(KSTEPS):
        parts = _spmm_kernel(ei, u8.reshape(NP, D))
        parts8 = parts.reshape(NC, R8, 128)
        if k < KSTEPS - 1:
            u8 = _combine_mid(parts8, inn, onn, h08)
        else:
            h8 = _combine_final(parts8, inn, h08)
    return h8.reshape(NP, D)[:N]
